# baseline (speedup 1.0000x reference)
import jax
import jax.numpy as jnp
from jax.experimental import pallas as pl
from jax.experimental.pallas import tpu as pltpu

_ISQRT2 = 0.7071067811865476
_K = 512


def _gcn_pool_body(v_ref, k2_ref, b_ref, out_ref, hlast_ref):
    i = pl.program_id(0)

    @pl.when(i == 0)
    def _init():
        hlast_ref[...] = jnp.zeros_like(hlast_ref)

    v = v_ref[...]
    k, fd = v.shape
    c = b_ref.shape[1]
    h = jnp.dot(v.astype(jnp.bfloat16), k2_ref[...],
                preferred_element_type=jnp.float32)
    carry = hlast_ref[...]
    hprev = jnp.concatenate([carry, h[:-1]], axis=0)
    hlast_ref[...] = h[-1:]
    g = jax.lax.broadcasted_iota(jnp.int32, (k, 1), 0) + i * k
    alpha = jnp.where(g == 0, 0.0, jnp.where(g == 1, _ISQRT2, 0.5))
    beta = jnp.where(g == 0, 1.0, 0.5)
    mixed = alpha.astype(jnp.float32) * hprev + beta.astype(jnp.float32) * h
    pooled = mixed[:, :c]
    for a in range(1, fd // c):
        pooled = jnp.maximum(pooled, mixed[:, a * c:(a + 1) * c])
    out_ref[...] = pooled + b_ref[...]


def _fc_body(p_ref, w1_ref, b1_ref, w2_ref, b2_ref, out_ref):
    p = p_ref[...]
    h1 = jnp.dot(p, w1_ref[...], preferred_element_type=jnp.float32)
    h1 = jnp.maximum(h1 + b1_ref[...], 0.0)
    o = jnp.dot(h1, w2_ref[...], preferred_element_type=jnp.float32)
    out_ref[...] = jax.nn.sigmoid(o + b2_ref[...])


def kernel(videos, W_gcn, b_gcn, W1, b1, W2, b2):
    B, V, F, D = videos.shape
    N = B * V
    C = W_gcn.shape[1]
    vp = videos.reshape(N, F * D)

    eye = jnp.eye(D, dtype=W_gcn.dtype)
    k2 = (eye[None, :, :, None] * W_gcn[:, None, None, :]).reshape(
        F * D, D * C).astype(jnp.bfloat16)

    pooled = pl.pallas_call(
        _gcn_pool_body,
        grid=(N // _K,),
        in_specs=[
            pl.BlockSpec((_K, F * D), lambda i: (i, 0)),
            pl.BlockSpec((F * D, D * C), lambda i: (0, 0)),
            pl.BlockSpec((1, C), lambda i: (0, 0)),
        ],
        out_specs=pl.BlockSpec((_K, C), lambda i: (i, 0)),
        out_shape=jax.ShapeDtypeStruct((N, C), jnp.float32),
        scratch_shapes=[pltpu.VMEM((1, D * C), jnp.float32)],
    )(vp, k2, b_gcn.reshape(1, C))

    out = pl.pallas_call(
        _fc_body,
        out_shape=jax.ShapeDtypeStruct((B, W2.shape[1]), jnp.float32),
    )(pooled.reshape(B, V * C), W1, b1.reshape(1, -1), W2,
      b2.reshape(1, -1))
    return out.reshape(B, V, F)


# speedup vs baseline: 1.5131x; 1.5131x over previous
import jax
import jax.numpy as jnp
from jax.experimental import pallas as pl
from jax.experimental.pallas import tpu as pltpu

_ISQRT2 = 0.7071067811865476


def _gcn_pool_body(v_ref, w_ref, b_ref, out_ref, hlast_ref):
    i = pl.program_id(0)

    @pl.when(i == 0)
    def _init():
        hlast_ref[...] = jnp.zeros_like(hlast_ref)

    v = v_ref[...]
    k, f, d = v.shape
    w = w_ref[...]
    c = w.shape[1]
    vt = jnp.swapaxes(v, 1, 2)
    h = jnp.dot(vt.reshape(k * d, f), w, preferred_element_type=jnp.float32)
    h = h.reshape(k, d, c)
    carry = hlast_ref[...]
    hprev = jnp.concatenate([carry, h[:-1]], axis=0)
    hlast_ref[...] = h[-1:]
    g = jax.lax.broadcasted_iota(jnp.int32, (k, 1, 1), 0) + i * k
    alpha = jnp.where(g == 0, 0.0, jnp.where(g == 1, _ISQRT2, 0.5))
    beta = jnp.where(g == 0, 1.0, 0.5)
    mixed = alpha.astype(jnp.float32) * hprev + beta.astype(jnp.float32) * h
    pooled = jnp.max(mixed, axis=1)
    out_ref[...] = pooled + b_ref[...]


def _fc_body(p_ref, w1_ref, b1_ref, w2_ref, b2_ref, out_ref):
    p = p_ref[...]
    h1 = jnp.dot(p, w1_ref[...], preferred_element_type=jnp.float32)
    h1 = jnp.maximum(h1 + b1_ref[...], 0.0)
    o = jnp.dot(h1, w2_ref[...], preferred_element_type=jnp.float32)
    out_ref[...] = jax.nn.sigmoid(o + b2_ref[...])


def kernel(videos, W_gcn, b_gcn, W1, b1, W2, b2):
    B, V, F, D = videos.shape
    N = B * V
    C = W_gcn.shape[1]
    K = 512
    v2 = videos.reshape(N, F, D)

    pooled = pl.pallas_call(
        _gcn_pool_body,
        grid=(N // K,),
        in_specs=[
            pl.BlockSpec((K, F, D), lambda i: (i, 0, 0)),
            pl.BlockSpec((F, C), lambda i: (0, 0)),
            pl.BlockSpec((1, C), lambda i: (0, 0)),
        ],
        out_specs=pl.BlockSpec((K, C), lambda i: (i, 0)),
        out_shape=jax.ShapeDtypeStruct((N, C), jnp.float32),
        scratch_shapes=[pltpu.VMEM((1, D, C), jnp.float32)],
    )(v2, W_gcn, b_gcn.reshape(1, C))

    out = pl.pallas_call(
        _fc_body,
        out_shape=jax.ShapeDtypeStruct((B, W2.shape[1]), jnp.float32),
    )(pooled.reshape(B, N // B * C), W1, b1.reshape(1, -1), W2,
      b2.reshape(1, -1))
    return out.reshape(B, V, F)


# speedup vs baseline: 1.5186x; 1.0036x over previous
import jax
import jax.numpy as jnp
from jax.experimental import pallas as pl
from jax.experimental.pallas import tpu as pltpu

_ISQRT2 = 0.7071067811865476


def _gcn_pool_body(v_ref, w_ref, b_ref, out_ref, hlast_ref):
    i = pl.program_id(0)

    @pl.when(i == 0)
    def _init():
        hlast_ref[...] = jnp.zeros_like(hlast_ref)

    v = v_ref[...]
    k, f, d = v.shape
    w = w_ref[...]
    c = w.shape[1]
    vt = jnp.swapaxes(v, 1, 2)
    h = jnp.dot(vt.reshape(k * d, f), w, preferred_element_type=jnp.float32)
    h = h.reshape(k, d, c)
    carry = hlast_ref[...]
    hprev = jnp.concatenate([carry, h[:-1]], axis=0)
    hlast_ref[...] = h[-1:]
    g = jax.lax.broadcasted_iota(jnp.int32, (k, 1, 1), 0) + i * k
    alpha = jnp.where(g == 0, 0.0, jnp.where(g == 1, _ISQRT2, 0.5))
    beta = jnp.where(g == 0, 1.0, 0.5)
    mixed = alpha.astype(jnp.float32) * hprev + beta.astype(jnp.float32) * h
    pooled = jnp.max(mixed, axis=1)
    out_ref[...] = pooled + b_ref[...]


def _fc_body(p_ref, w1_ref, b1_ref, w2_ref, b2_ref, out_ref):
    p = p_ref[...]
    h1 = jnp.dot(p, w1_ref[...], preferred_element_type=jnp.float32)
    h1 = jnp.maximum(h1 + b1_ref[...], 0.0)
    o = jnp.dot(h1, w2_ref[...], preferred_element_type=jnp.float32)
    out_ref[...] = jax.nn.sigmoid(o + b2_ref[...])


def kernel(videos, W_gcn, b_gcn, W1, b1, W2, b2):
    B, V, F, D = videos.shape
    N = B * V
    C = W_gcn.shape[1]
    K = 1024
    v2 = videos.reshape(N, F, D)

    pooled = pl.pallas_call(
        _gcn_pool_body,
        grid=(N // K,),
        in_specs=[
            pl.BlockSpec((K, F, D), lambda i: (i, 0, 0)),
            pl.BlockSpec((F, C), lambda i: (0, 0)),
            pl.BlockSpec((1, C), lambda i: (0, 0)),
        ],
        out_specs=pl.BlockSpec((K, C), lambda i: (i, 0)),
        out_shape=jax.ShapeDtypeStruct((N, C), jnp.float32),
        scratch_shapes=[pltpu.VMEM((1, D, C), jnp.float32)],
    )(v2, W_gcn, b_gcn.reshape(1, C))

    out = pl.pallas_call(
        _fc_body,
        out_shape=jax.ShapeDtypeStruct((B, W2.shape[1]), jnp.float32),
    )(pooled.reshape(B, N // B * C), W1, b1.reshape(1, -1), W2,
      b2.reshape(1, -1))
    return out.reshape(B, V, F)
